# baseline (speedup 1.0000x reference)
import jax
import jax.numpy as jnp
from jax.experimental import pallas as pl
from jax.experimental.pallas import tpu as pltpu

S = 2048
D = 1024


def _body(bos_ref, idx_ref, embeds_ref, speech_ref, pos_ref, out_ref,
          row_a, row_b, row_c, sem_bulk, sem_a, sem_b, sem_c0, sem_c1):
    bulk = pltpu.make_async_copy(embeds_ref, out_ref.at[:, pl.ds(0, S), :],
                                 sem_bulk)
    bulk.start()

    tok = bos_ref[0, 0]
    ix = idx_ref[0]
    cp_a = pltpu.make_async_copy(speech_ref.at[pl.ds(tok, 1), :], row_a, sem_a)
    cp_b = pltpu.make_async_copy(pos_ref.at[pl.ds(ix, 1), :], row_b, sem_b)
    cp_a.start()
    cp_b.start()
    cp_a.wait()
    cp_b.wait()
    row_c[...] = row_a[...] + row_b[...]

    cp0 = pltpu.make_async_copy(row_c, out_ref.at[0, pl.ds(S, 1), :], sem_c0)
    cp1 = pltpu.make_async_copy(row_c, out_ref.at[1, pl.ds(S, 1), :], sem_c1)
    cp0.start()
    cp1.start()
    cp0.wait()
    cp1.wait()
    bulk.wait()


def kernel(bos_token, embeds, idx, speech_emb, pos_emb):
    out = pl.pallas_call(
        _body,
        out_shape=jax.ShapeDtypeStruct((2, S + 1, D), jnp.float32),
        in_specs=[
            pl.BlockSpec(memory_space=pltpu.SMEM),
            pl.BlockSpec(memory_space=pltpu.SMEM),
            pl.BlockSpec(memory_space=pl.ANY),
            pl.BlockSpec(memory_space=pl.ANY),
            pl.BlockSpec(memory_space=pl.ANY),
        ],
        out_specs=pl.BlockSpec(memory_space=pl.ANY),
        scratch_shapes=[
            pltpu.VMEM((1, D), jnp.float32),
            pltpu.VMEM((1, D), jnp.float32),
            pltpu.VMEM((1, D), jnp.float32),
            pltpu.SemaphoreType.DMA,
            pltpu.SemaphoreType.DMA,
            pltpu.SemaphoreType.DMA,
            pltpu.SemaphoreType.DMA,
            pltpu.SemaphoreType.DMA,
        ],
    )(bos_token, idx, embeds, speech_emb, pos_emb)
    return out
